# Initial kernel scaffold; baseline (speedup 1.0000x reference)
import jax
import jax.numpy as jnp
from jax.experimental import pallas as pl


def kernel(images, w0, w1, w2, w3, w4, w5, w6, w7, w8, w9, w10, w11, w12, w13, w14, w15, w16, w17, w18, w19, w20, w21, w22, w23, w24, w25, w26, w27, w28, w29, w30, w31, w32, w33, w34, w35, w36, w37, w38, w39, w40, w41, w42, w43, w44, w45, w46, w47, w48, w49, w50, w51, w52, w53, w54, w55, w56, w57, w58, w59, w60, w61, w62, w63, w64, w65, w66, w67, w68, w69, w70, w71, w72, w73, w74, w75, w76, w77, w78, w79, w80, w81, w82, w83, w84, w85, w86, w87, w88, w89, w90, w91, w92, w93, w94, w95, w96, w97, w98, w99, w100, w101, w102, w103, w104, w105, w106, w107, w108, w109, w110, w111, w112, w113, w114, w115, w116, w117, w118, w119, w120, w121, w122, w123, w124, w125, w126, w127, w128, w129, w130, w131, w132, w133, w134, w135, w136, w137, w138, w139, w140, w141, w142, w143, w144, w145, w146, w147, w148, w149, w150, w151, w152, w153, w154, w155, w156, w157, w158, w159, w160, w161, w162, w163, w164, w165, w166, w167, w168, w169, w170, w171, w172, w173, w174, w175, w176, w177, w178, w179, w180, w181, w182, w183, w184, w185, w186, w187, w188, w189, w190, w191, w192, w193, w194, w195, w196, w197, w198, w199, w200, w201, w202, w203, w204, w205, w206, w207, w208, w209, w210, w211, w212, w213, w214, w215, w216, w217, w218, w219, w220, w221, w222, w223, w224, w225, w226, w227, w228, w229, w230, w231, w232, w233, w234, w235, w236, w237, w238, w239, w240, w241, w242, w243, w244, w245, w246, w247, w248, w249, w250, w251, w252, w253, w254, w255, w256, w257, w258, w259, w260, w261, w262, w263, w264, w265, w266, w267, w268, w269, w270, w271, w272, w273, w274, w275, w276, w277, w278, w279, w280, w281, w282, w283, w284, w285, w286, w287, w288, w289, w290, w291, w292, w293, w294, w295, w296, w297, w298, w299, w300, w301, w302, w303, w304, w305, w306, w307, w308, w309, w310, w311, w312, w313, w314, w315, w316, w317, w318, w319, w320, w321, w322, w323, w324, w325, w326, w327, w328, w329, w330, w331, w332, w333, w334, w335, w336, w337, w338, w339, w340, w341, w342, w343, w344, w345, w346, w347, w348, w349, w350, w351, w352, w353, w354, w355, w356, w357, w358, w359, w360, w361, w362, w363, w364, w365, w366, w367, w368, w369, w370, w371, w372, w373, w374, w375, w376, w377, w378, w379, w380, w381, w382, w383, w384, w385, w386, w387, w388, w389, w390, w391, w392, w393, w394, w395, w396, w397, w398, w399, w400, w401, w402, w403, w404, w405, w406, w407, w408, w409, w410, w411, w412, w413, w414, w415, w416, w417, w418, w419, w420, w421, w422, w423, w424, w425, w426, w427, w428, w429, w430, w431, w432, w433, w434, w435, w436, w437, w438, w439, w440, w441, w442, w443, w444, w445, w446, w447, w448, w449, w450, w451, w452, w453, w454, w455, w456, w457, w458, w459, w460, w461, w462, w463, w464, w465, w466, w467):
    raise NotImplementedError("write your pallas kernel here")


# speedup vs baseline: 1.3087x; 1.3087x over previous
import functools

import jax
import jax.numpy as jnp
from jax.experimental import pallas as pl
from jax.experimental.pallas import tpu as pltpu


def _ceil_to(x, m):
    return ((x + m - 1) // m) * m


def _tile_n(n):
    if n % 256 == 0:
        return 256
    if n % 128 == 0:
        return 128
    return n


def _tile_m(m):
    if m <= 1024:
        return m
    for d in range(512, 7, -8):
        if m % d == 0:
            return d
    return 256


def _gemm_kernel(a_ref, b_ref, s_ref, c_ref, o_ref, *, relu):
    acc = jnp.dot(a_ref[...], b_ref[...], preferred_element_type=jnp.float32)
    y = acc * s_ref[...] + c_ref[...]
    if relu:
        y = jnp.maximum(y, 0.0)
    o_ref[...] = y.astype(o_ref.dtype)


def _gemm_res_kernel(a_ref, b_ref, s_ref, c_ref, r_ref, o_ref, *, relu):
    acc = jnp.dot(a_ref[...], b_ref[...], preferred_element_type=jnp.float32)
    y = acc * s_ref[...] + c_ref[...] + r_ref[...].astype(jnp.float32)
    if relu:
        y = jnp.maximum(y, 0.0)
    o_ref[...] = y.astype(o_ref.dtype)


def _gemm_bn(a, b, scale, bias, residual=None, *, relu,
             out_dtype=jnp.bfloat16):
    m, k = a.shape
    n = b.shape[1]
    tn = _tile_n(n)
    tm = _tile_m(m)
    mp = _ceil_to(m, tm)

    a_p = a.astype(jnp.bfloat16)
    if mp != m:
        a_p = jnp.pad(a_p, ((0, mp - m), (0, 0)))

    in_specs = [
        pl.BlockSpec((tm, k), lambda j, i: (i, 0)),
        pl.BlockSpec((k, tn), lambda j, i: (0, j)),
        pl.BlockSpec((1, tn), lambda j, i: (0, j)),
        pl.BlockSpec((1, tn), lambda j, i: (0, j)),
    ]
    args = [a_p, b, scale, bias]
    if residual is None:
        body = functools.partial(_gemm_kernel, relu=relu)
    else:
        r_p = residual.astype(jnp.bfloat16)
        if mp != m:
            r_p = jnp.pad(r_p, ((0, mp - m), (0, 0)))
        in_specs.append(pl.BlockSpec((tm, tn), lambda j, i: (i, j)))
        args.append(r_p)
        body = functools.partial(_gemm_res_kernel, relu=relu)

    out = pl.pallas_call(
        body,
        out_shape=jax.ShapeDtypeStruct((mp, n), out_dtype),
        grid=(n // tn, mp // tm),
        in_specs=in_specs,
        out_specs=pl.BlockSpec((tm, tn), lambda j, i: (i, j)),
        compiler_params=pltpu.CompilerParams(
            dimension_semantics=("parallel", "parallel")),
    )(*args)
    return out if mp == m else out[:m]


def _c3_kernel(x_ref, w_ref, s_ref, c_ref, o_ref, *, g, h, w, relu):
    cin = x_ref.shape[3]
    wp = w + 2
    hp = h + 3
    tn = o_ref.shape[3]
    xf = x_ref[...].reshape(g * hp * wp, cin)
    acc = jnp.zeros((g, h * wp, tn), jnp.float32)
    for dh in range(3):
        for dw in range(3):
            y = jnp.dot(xf, w_ref[3 * dh + dw],
                        preferred_element_type=jnp.float32)
            y = y.reshape(g, hp * wp, tn)
            off = dh * wp + dw
            acc = acc + y[:, off:off + h * wp, :]
    y = acc * s_ref[...].reshape(1, 1, tn) + c_ref[...].reshape(1, 1, tn)
    if relu:
        y = jnp.maximum(y, 0.0)
    o_ref[...] = y.reshape(g, h, wp, tn)[:, :, :w, :].astype(o_ref.dtype)


def _conv3x3_bn(x, w9, scale, bias, *, relu=True):
    n, h, w, cin = x.shape
    cout = w9.shape[2]
    tn = _tile_n(cout)
    hp, wp = h + 3, w + 2
    area = hp * wp
    g = 1
    while g < n and g * area < 1024:
        g *= 2
    xp = jnp.pad(x, ((0, 0), (1, 2), (1, 1), (0, 0)))
    body = functools.partial(_c3_kernel, g=g, h=h, w=w, relu=relu)
    return pl.pallas_call(
        body,
        out_shape=jax.ShapeDtypeStruct((n, h, w, cout), jnp.bfloat16),
        grid=(cout // tn, n // g),
        in_specs=[
            pl.BlockSpec((g, hp, wp, cin), lambda j, i: (i, 0, 0, 0)),
            pl.BlockSpec((9, cin, tn), lambda j, i: (0, 0, j)),
            pl.BlockSpec((1, tn), lambda j, i: (0, j)),
            pl.BlockSpec((1, tn), lambda j, i: (0, j)),
        ],
        out_specs=pl.BlockSpec((g, h, w, tn), lambda j, i: (i, 0, 0, j)),
        compiler_params=pltpu.CompilerParams(
            dimension_semantics=("parallel", "parallel")),
    )(xp, w9, scale, bias)


def _pool_kernel(ee_ref, eo_ref, oe_ref, oo_ref, o_ref):
    ho, wo = o_ref.shape[1], o_ref.shape[2]
    ee = ee_ref[...]
    eo = eo_ref[...]
    oe = oe_ref[...]
    oo = oo_ref[...]
    top = jnp.maximum(jnp.maximum(ee[:, :ho, :wo], eo[:, :ho, :]),
                      ee[:, :ho, 1:])
    mid = jnp.maximum(jnp.maximum(oe[:, :, :wo], oo), oe[:, :, 1:])
    bot = jnp.maximum(jnp.maximum(ee[:, 1:, :wo], eo[:, 1:, :]),
                      ee[:, 1:, 1:])
    o_ref[...] = jnp.maximum(jnp.maximum(top, mid), bot)


def _maxpool_3x3_s2(x):
    n, h, w, c = x.shape
    ho = (h + 2 - 3) // 2 + 1
    wo = (w + 2 - 3) // 2 + 1
    xp = jnp.pad(x, ((0, 0), (1, 1), (1, 1), (0, 0)),
                 constant_values=-jnp.inf)
    ee = xp[:, 0::2, 0::2, :][:, :ho + 1, :wo + 1, :]
    eo = xp[:, 0::2, 1::2, :][:, :ho + 1, :wo, :]
    oe = xp[:, 1::2, 0::2, :][:, :ho, :wo + 1, :]
    oo = xp[:, 1::2, 1::2, :][:, :ho, :wo, :]
    g = 4
    return pl.pallas_call(
        _pool_kernel,
        out_shape=jax.ShapeDtypeStruct((n, ho, wo, c), x.dtype),
        grid=(n // g,),
        in_specs=[
            pl.BlockSpec((g, ho + 1, wo + 1, c), lambda i: (i, 0, 0, 0)),
            pl.BlockSpec((g, ho + 1, wo, c), lambda i: (i, 0, 0, 0)),
            pl.BlockSpec((g, ho, wo + 1, c), lambda i: (i, 0, 0, 0)),
            pl.BlockSpec((g, ho, wo, c), lambda i: (i, 0, 0, 0)),
        ],
        out_specs=pl.BlockSpec((g, ho, wo, c), lambda i: (i, 0, 0, 0)),
        compiler_params=pltpu.CompilerParams(
            dimension_semantics=("parallel",)),
    )(ee, eo, oe, oo)


def _head_kernel(x_ref, w_ref, s_ref, c_ref, o_ref):
    hw = x_ref.shape[1]
    feat = (jnp.sum(x_ref[...].astype(jnp.float32), axis=1)
            * (1.0 / hw)).astype(jnp.bfloat16)
    acc = jnp.dot(feat, w_ref[...], preferred_element_type=jnp.float32)
    o_ref[...] = acc * s_ref[...] + c_ref[...]


def _head(x, fc_w, fc_scale, fc_bias):
    n, h, w, c = x.shape
    hw = h * w
    ncp = fc_w.shape[1]
    xr = x.reshape(n, hw, c)
    return pl.pallas_call(
        _head_kernel,
        out_shape=jax.ShapeDtypeStruct((n, ncp), jnp.float32),
        grid=(1,),
        in_specs=[
            pl.BlockSpec((n, hw, c), lambda i: (0, 0, 0)),
            pl.BlockSpec((c, ncp), lambda i: (0, 0)),
            pl.BlockSpec((1, ncp), lambda i: (0, 0)),
            pl.BlockSpec((1, ncp), lambda i: (0, 0)),
        ],
        out_specs=pl.BlockSpec((n, ncp), lambda i: (0, 0)),
        compiler_params=pltpu.CompilerParams(
            dimension_semantics=("arbitrary",)),
    )(xr, fc_w, fc_scale, fc_bias)


def _patches(x, kh, kw, stride, padding):
    n, h, w, c = x.shape
    ho = (h + 2 * padding - kh) // stride + 1
    wo = (w + 2 * padding - kw) // stride + 1
    if kh == 1 and kw == 1 and padding == 0:
        xs = x[:, ::stride, ::stride, :] if stride != 1 else x
        return xs.reshape(n * ho * wo, c), ho, wo
    xp = jnp.pad(x, ((0, 0), (padding, padding), (padding, padding), (0, 0)))
    cols = []
    for ih in range(kh):
        for iw in range(kw):
            cols.append(xp[:, ih:ih + (ho - 1) * stride + 1:stride,
                           iw:iw + (wo - 1) * stride + 1:stride, :])
    pat = jnp.stack(cols, axis=3).reshape(n * ho * wo, kh * kw * c)
    return pat, ho, wo


def _conv_bn(x, cp, bn, *, stride, padding, relu, residual=None):
    scale, bias = bn
    if "w9" in cp:
        return _conv3x3_bn(x, cp["w9"], scale, bias, relu=relu)
    n = x.shape[0]
    cout = cp["cout"]
    pat, ho, wo = _patches(x, cp["kh"], cp["kw"], stride, padding)
    r2 = None if residual is None else residual.reshape(n * ho * wo, cout)
    y = _gemm_bn(pat, cp["wm"], scale, bias, r2, relu=relu)
    return y.reshape(n, ho, wo, cout)


def _conv_t(cout, kh, kw, fuse3x3=False):
    p = {"kh": kh, "kw": kw, "cout": cout}
    if fuse3x3:
        p["w9"] = 0.0
    else:
        p["wm"] = 0.0
    return p


def _bn_t():
    return (0.0, 0.0)


def _param_template():
    params = {
        "conv1": _conv_t(64, 7, 7),
        "bn1": _bn_t(),
        "num_classes": 1000,
        "fc_wm": 0.0,
        "fc_scale": 0.0,
        "fc_bias": 0.0,
    }
    blocks_per_layer = (3, 8, 36, 3)
    planes_per_layer = (64, 128, 256, 512)
    inplanes = 64
    layers = []
    for li, (nblocks, planes) in enumerate(zip(blocks_per_layer,
                                               planes_per_layer)):
        layer_stride = 1 if li == 0 else 2
        blocks = []
        for bi in range(nblocks):
            s = layer_stride if bi == 0 else 1
            p = {
                "conv1": _conv_t(planes, 1, 1),
                "bn1": _bn_t(),
                "conv2": _conv_t(planes, 3, 3, fuse3x3=(s == 1)),
                "bn2": _bn_t(),
                "conv3": _conv_t(planes * 4, 1, 1),
                "bn3": _bn_t(),
                "stride": s,
            }
            if s != 1 or inplanes != planes * 4:
                p["down_conv"] = _conv_t(planes * 4, 1, 1)
                p["down_bn"] = _bn_t()
            blocks.append(p)
            inplanes = planes * 4
        layers.append(blocks)
    params["layers"] = layers
    return params


def _block_fwd(x, p):
    stride = p["stride"]
    out = _conv_bn(x, p["conv1"], p["bn1"], stride=1, padding=0, relu=True)
    out = _conv_bn(out, p["conv2"], p["bn2"], stride=stride, padding=1,
                   relu=True)
    if "down_conv" in p:
        identity = _conv_bn(x, p["down_conv"], p["down_bn"], stride=stride,
                            padding=0, relu=False)
    else:
        identity = x
    return _conv_bn(out, p["conv3"], p["bn3"], stride=1, padding=0,
                    relu=True, residual=identity)


def kernel(images, *ws):
    leaves, treedef = jax.tree_util.tree_flatten(_param_template())
    it = iter(ws)
    new_leaves = [leaf if isinstance(leaf, int) else next(it)
                  for leaf in leaves]
    params = jax.tree_util.tree_unflatten(treedef, new_leaves)

    x = jnp.transpose(images.astype(jnp.float32),
                      (0, 2, 3, 1)).astype(jnp.bfloat16)
    x = _conv_bn(x, params["conv1"], params["bn1"], stride=2, padding=3,
                 relu=True)
    x = _maxpool_3x3_s2(x)
    for blocks in params["layers"]:
        for blk in blocks:
            x = _block_fwd(x, blk)
    logits = _head(x, params["fc_wm"], params["fc_scale"], params["fc_bias"])
    return logits[:, :params["num_classes"]]
